# Initial kernel scaffold; baseline (speedup 1.0000x reference)
import functools

import jax
import jax.numpy as jnp
from jax import lax
from jax.experimental import pallas as pl
from jax.experimental.pallas import tpu as pltpu
from jax.experimental.pallas import tpu_sc as plsc

NC = 2
NS = 16
NW = NC * NS
L = 16
B = 128
CH = 16
CHUNK = CH * B
BN = 1024


def _sc_scatter_pass(src2d, dst2d, tables, n_pad):
    nt = len(tables)
    rows_w = src2d.shape[0] // NW
    n_chunks = rows_w // CH
    tile_rows = n_pad // NS
    mesh = plsc.VectorSubcoreMesh(core_axis_name="c", subcore_axis_name="s")

    @functools.partial(
        pl.kernel,
        out_type=jax.ShapeDtypeStruct((nt, NC, n_pad, L), jnp.float32),
        mesh=mesh,
        scratch_types=[
            pltpu.VMEM((CH, B), jnp.int32),
            pltpu.VMEM((CH, B), jnp.int32),
            pltpu.VMEM((CHUNK, L), jnp.float32),
            pltpu.VMEM_SHARED((n_pad, L), jnp.float32),
            pltpu.VMEM((64, L), jnp.float32),
            pltpu.SemaphoreType.DMA,
        ],
    )
    def k(src_hbm, dst_hbm, *rest):
        tabs = rest[:nt]
        out_hbm, src_v, dst_v, rows_v, acc, zbuf, sem = rest[nt:]
        c = lax.axis_index("c")
        s = lax.axis_index("s")
        wid = s * NC + c
        base_row = wid * rows_w
        t0 = s * tile_rows

        zeros16 = jnp.zeros((L,), jnp.float32)
        for r in range(64):
            zbuf[r, :] = zeros16

        for f in range(nt):
            def zero_body(i, _):
                pltpu.sync_copy(zbuf, acc.at[pl.ds(t0 + i * 64, 64)])
                return 0
            lax.fori_loop(0, tile_rows // 64, zero_body, 0)
            plsc.subcore_barrier()

            def body(i, _):
                r0 = base_row + i * CH
                pltpu.sync_copy(dst_hbm.at[pl.ds(r0, CH)], dst_v)
                pltpu.sync_copy(src_hbm.at[pl.ds(r0, CH)], src_v)
                cps = []
                for j in range(CH):
                    cps.append(pltpu.async_copy(
                        tabs[f].at[src_v.at[j]],
                        rows_v.at[pl.ds(j * B, B)], sem))
                for cp in cps:
                    cp.wait()
                for j in range(CH):
                    pltpu.sync_copy(rows_v.at[pl.ds(j * B, B)],
                                    acc.at[dst_v.at[j]], add=True)
                return 0
            lax.fori_loop(0, n_chunks, body, 0)
            plsc.subcore_barrier()

            pltpu.sync_copy(acc.at[pl.ds(t0, tile_rows)],
                            out_hbm.at[f, c, pl.ds(t0, tile_rows)])

    return k(src2d, dst2d, *tables)


def _sc_degree_pass(dst2d, n_pad):
    rows_w = dst2d.shape[0] // NW
    n_chunks = rows_w // CH
    tile_rows = n_pad // NS
    mesh = plsc.VectorSubcoreMesh(core_axis_name="c", subcore_axis_name="s")

    @functools.partial(
        pl.kernel,
        out_type=jax.ShapeDtypeStruct((NC, n_pad, L), jnp.float32),
        mesh=mesh,
        scratch_types=[
            pltpu.VMEM((CH, B), jnp.int32),
            pltpu.VMEM((B, L), jnp.float32),
            pltpu.VMEM_SHARED((n_pad, L), jnp.float32),
            pltpu.VMEM((64, L), jnp.float32),
        ],
    )
    def k(dst_hbm, out_hbm, dst_v, ones_v, acc, zbuf):
        c = lax.axis_index("c")
        s = lax.axis_index("s")
        wid = s * NC + c
        base_row = wid * rows_w
        t0 = s * tile_rows

        zeros16 = jnp.zeros((L,), jnp.float32)
        ones16 = jnp.ones((L,), jnp.float32)
        for r in range(64):
            zbuf[r, :] = zeros16
        for r in range(B):
            ones_v[r, :] = ones16

        def zero_body(i, _):
            pltpu.sync_copy(zbuf, acc.at[pl.ds(t0 + i * 64, 64)])
            return 0
        lax.fori_loop(0, tile_rows // 64, zero_body, 0)
        plsc.subcore_barrier()

        def body(i, _):
            r0 = base_row + i * CH
            pltpu.sync_copy(dst_hbm.at[pl.ds(r0, CH)], dst_v)
            for j in range(CH):
                pltpu.sync_copy(ones_v, acc.at[dst_v.at[j]], add=True)
            return 0
        lax.fori_loop(0, n_chunks, body, 0)
        plsc.subcore_barrier()

        pltpu.sync_copy(acc.at[pl.ds(t0, tile_rows)],
                        out_hbm.at[c, pl.ds(t0, tile_rows)])

    return k(dst2d)


def _tc_prep(degp, xpad, n_pad):
    grid = n_pad // BN

    def body(degp_ref, x_ref, dinv_ref, g1_ref):
        deg = 1.0 + degp_ref[0, :, 0:1] + degp_ref[1, :, 0:1]
        dinv = lax.rsqrt(deg)
        dinv_ref[...] = dinv
        g1_ref[...] = x_ref[...] * dinv

    return pl.pallas_call(
        body,
        grid=(grid,),
        in_specs=[
            pl.BlockSpec((NC, BN, L), lambda i: (0, i, 0)),
            pl.BlockSpec((BN, L), lambda i: (i, 0)),
        ],
        out_specs=[
            pl.BlockSpec((BN, 1), lambda i: (i, 0)),
            pl.BlockSpec((BN, L), lambda i: (i, 0)),
        ],
        out_shape=[
            jax.ShapeDtypeStruct((n_pad, 1), jnp.float32),
            jax.ShapeDtypeStruct((n_pad, L), jnp.float32),
        ],
    )(degp, xpad)


def _tc_mid(p1, g1, dinv, W1p, b1, W2, b2, n_pad):
    grid = n_pad // BN
    H = W2.shape[0]

    def body(p_ref, g1_ref, dinv_ref, w1_ref, b1_ref, w2_ref, *out_refs):
        dinv = dinv_ref[...]
        ax = (p_ref[0] + p_ref[1] + g1_ref[...]) * dinv
        h1 = jnp.maximum(
            jnp.dot(ax, w1_ref[...], preferred_element_type=jnp.float32)
            + b1_ref[...], 0.0)
        g2 = jnp.dot(h1, w2_ref[...], preferred_element_type=jnp.float32) * dinv
        for f in range(4):
            out_refs[f][...] = g2[:, f * L:(f + 1) * L]

    return pl.pallas_call(
        body,
        grid=(grid,),
        in_specs=[
            pl.BlockSpec((NC, BN, L), lambda i: (0, i, 0)),
            pl.BlockSpec((BN, L), lambda i: (i, 0)),
            pl.BlockSpec((BN, 1), lambda i: (i, 0)),
            pl.BlockSpec((L, H), lambda i: (0, 0)),
            pl.BlockSpec((1, H), lambda i: (0, 0)),
            pl.BlockSpec((H, H), lambda i: (0, 0)),
        ],
        out_specs=[pl.BlockSpec((BN, L), lambda i: (i, 0)) for _ in range(4)],
        out_shape=[jax.ShapeDtypeStruct((n_pad, L), jnp.float32)
                   for _ in range(4)],
    )(p1, g1, dinv, W1p, b1.reshape(1, H), W2)


def _tc_final(agg, g2s, dinv, b2, Wl, bl, n_pad):
    grid = n_pad // BN
    H = Wl.shape[0]
    OUT = Wl.shape[1]

    def body(agg_ref, g0, g1, g2, g3, dinv_ref, b2_ref, wl_ref, bl_ref, out_ref):
        dinv = dinv_ref[...]
        gs = (g0, g1, g2, g3)
        parts = [agg_ref[f, 0] + agg_ref[f, 1] + gs[f][...] for f in range(4)]
        pre = jnp.concatenate(parts, axis=1) * dinv + b2_ref[...]
        h2 = jnp.maximum(pre, 0.0)
        out_ref[...] = (jnp.dot(h2, wl_ref[...],
                                preferred_element_type=jnp.float32)
                        + bl_ref[...])

    return pl.pallas_call(
        body,
        grid=(grid,),
        in_specs=[
            pl.BlockSpec((4, NC, BN, L), lambda i: (0, 0, i, 0)),
            pl.BlockSpec((BN, L), lambda i: (i, 0)),
            pl.BlockSpec((BN, L), lambda i: (i, 0)),
            pl.BlockSpec((BN, L), lambda i: (i, 0)),
            pl.BlockSpec((BN, L), lambda i: (i, 0)),
            pl.BlockSpec((BN, 1), lambda i: (i, 0)),
            pl.BlockSpec((1, H), lambda i: (0, 0)),
            pl.BlockSpec((H, OUT), lambda i: (0, 0)),
            pl.BlockSpec((1, OUT), lambda i: (0, 0)),
        ],
        out_specs=pl.BlockSpec((BN, OUT), lambda i: (i, 0)),
        out_shape=jax.ShapeDtypeStruct((n_pad, OUT), jnp.float32),
    )(agg, *g2s, dinv, b2.reshape(1, H), Wl, bl.reshape(1, OUT))


def kernel(x, edge_index, W1, b1, W2, b2, Wl, bl):
    N, IN = x.shape
    E = edge_index.shape[1]

    n_pad = -(-(N + 1) // BN) * BN
    ew = -(-E // (NW * CHUNK)) * CHUNK
    Ep = NW * ew
    src = jnp.concatenate(
        [edge_index[0], jnp.zeros((Ep - E,), jnp.int32)]).reshape(Ep // B, B)
    dst = jnp.concatenate(
        [edge_index[1], jnp.full((Ep - E,), N, jnp.int32)]).reshape(Ep // B, B)

    xpad = jnp.zeros((n_pad, L), jnp.float32).at[:N, :IN].set(x)
    W1p = jnp.pad(W1, ((0, L - IN), (0, 0)))

    degp = _sc_degree_pass(dst, n_pad)
    dinv, g1 = _tc_prep(degp, xpad, n_pad)
    p1 = _sc_scatter_pass(src, dst, [g1], n_pad)[0]
    g2s = _tc_mid(p1, g1, dinv, W1p, b1, W2, b2, n_pad)
    agg = _sc_scatter_pass(src, dst, list(g2s), n_pad)
    out = _tc_final(agg, g2s, dinv, b2, Wl, bl, n_pad)
    return out[:N]


# speedup vs baseline: 18.1532x; 18.1532x over previous
import functools

import jax
import jax.numpy as jnp
from jax import lax
from jax.experimental import pallas as pl
from jax.experimental.pallas import tpu as pltpu
from jax.experimental.pallas import tpu_sc as plsc

NC = 2
NS = 16
NW = NC * NS
L = 16
B = 128
CH = 8
CHUNK = CH * B
BN = 1024


def _sc_scatter_pass(src2d, dst2d, tables, n_pad):
    nt = len(tables)
    rows_w = src2d.shape[0] // NW
    n_chunks = rows_w // CH
    tile_rows = n_pad // NS
    mesh = plsc.VectorSubcoreMesh(core_axis_name="c", subcore_axis_name="s")

    @functools.partial(
        pl.kernel,
        out_type=jax.ShapeDtypeStruct((nt, NC, n_pad, L), jnp.float32),
        mesh=mesh,
        scratch_types=[
            pltpu.VMEM((CH, B), jnp.int32),
            pltpu.VMEM((CH, B), jnp.int32),
            pltpu.VMEM((CHUNK, L), jnp.float32),
            pltpu.VMEM_SHARED((n_pad, L), jnp.float32),
            pltpu.VMEM((64, L), jnp.float32),
            pltpu.SemaphoreType.DMA,
        ],
        compiler_params=pltpu.CompilerParams(use_tc_tiling_on_sc=False),
    )
    def k(src_hbm, dst_hbm, *rest):
        tabs = rest[:nt]
        out_hbm, src_v, dst_v, rows_v, acc, zbuf, sem = rest[nt:]
        c = lax.axis_index("c")
        s = lax.axis_index("s")
        wid = s * NC + c
        base_row = wid * rows_w
        t0 = s * tile_rows

        zeros16 = jnp.zeros((L,), jnp.float32)
        for r in range(64):
            zbuf[r, :] = zeros16

        for f in range(nt):
            def zero_body(i, _):
                pltpu.sync_copy(zbuf, acc.at[pl.ds(t0 + i * 64, 64)])
                return 0
            lax.fori_loop(0, tile_rows // 64, zero_body, 0)
            plsc.subcore_barrier()

            def body(i, _):
                r0 = base_row + i * CH
                pltpu.sync_copy(dst_hbm.at[pl.ds(r0, CH)], dst_v)
                pltpu.sync_copy(src_hbm.at[pl.ds(r0, CH)], src_v)
                cps = []
                for j in range(CH):
                    cps.append(pltpu.async_copy(
                        tabs[f].at[src_v.at[j]],
                        rows_v.at[pl.ds(j * B, B)], sem))
                for cp in cps:
                    cp.wait()
                for j in range(CH):
                    pltpu.sync_copy(rows_v.at[pl.ds(j * B, B)],
                                    acc.at[dst_v.at[j]], add=True)
                return 0
            lax.fori_loop(0, n_chunks, body, 0)
            plsc.subcore_barrier()

            pltpu.sync_copy(acc.at[pl.ds(t0, tile_rows)],
                            out_hbm.at[f, c, pl.ds(t0, tile_rows)])

    return k(src2d, dst2d, *tables)


def _sc_degree_pass(dst2d, n_pad):
    rows_w = dst2d.shape[0] // NW
    n_chunks = rows_w // CH
    tile_rows = n_pad // NS
    mesh = plsc.VectorSubcoreMesh(core_axis_name="c", subcore_axis_name="s")

    @functools.partial(
        pl.kernel,
        out_type=jax.ShapeDtypeStruct((NC, n_pad, L), jnp.float32),
        mesh=mesh,
        scratch_types=[
            pltpu.VMEM((CH, B), jnp.int32),
            pltpu.VMEM((B, L), jnp.float32),
            pltpu.VMEM_SHARED((n_pad, L), jnp.float32),
            pltpu.VMEM((64, L), jnp.float32),
        ],
        compiler_params=pltpu.CompilerParams(use_tc_tiling_on_sc=False),
    )
    def k(dst_hbm, out_hbm, dst_v, ones_v, acc, zbuf):
        c = lax.axis_index("c")
        s = lax.axis_index("s")
        wid = s * NC + c
        base_row = wid * rows_w
        t0 = s * tile_rows

        zeros16 = jnp.zeros((L,), jnp.float32)
        ones16 = jnp.ones((L,), jnp.float32)
        for r in range(64):
            zbuf[r, :] = zeros16
        for r in range(B):
            ones_v[r, :] = ones16

        def zero_body(i, _):
            pltpu.sync_copy(zbuf, acc.at[pl.ds(t0 + i * 64, 64)])
            return 0
        lax.fori_loop(0, tile_rows // 64, zero_body, 0)
        plsc.subcore_barrier()

        def body(i, _):
            r0 = base_row + i * CH
            pltpu.sync_copy(dst_hbm.at[pl.ds(r0, CH)], dst_v)
            for j in range(CH):
                pltpu.sync_copy(ones_v, acc.at[dst_v.at[j]], add=True)
            return 0
        lax.fori_loop(0, n_chunks, body, 0)
        plsc.subcore_barrier()

        pltpu.sync_copy(acc.at[pl.ds(t0, tile_rows)],
                        out_hbm.at[c, pl.ds(t0, tile_rows)])

    return k(dst2d)


def _tc_prep(degp, xpad, n_pad):
    grid = n_pad // BN

    def body(degp_ref, x_ref, dinv_ref, g1_ref):
        deg = 1.0 + degp_ref[0, :, 0:1] + degp_ref[1, :, 0:1]
        dinv = lax.rsqrt(deg)
        dinv_ref[...] = dinv
        g1_ref[...] = x_ref[...] * dinv

    return pl.pallas_call(
        body,
        grid=(grid,),
        in_specs=[
            pl.BlockSpec((NC, BN, L), lambda i: (0, i, 0)),
            pl.BlockSpec((BN, L), lambda i: (i, 0)),
        ],
        out_specs=[
            pl.BlockSpec((BN, 1), lambda i: (i, 0)),
            pl.BlockSpec((BN, L), lambda i: (i, 0)),
        ],
        out_shape=[
            jax.ShapeDtypeStruct((n_pad, 1), jnp.float32),
            jax.ShapeDtypeStruct((n_pad, L), jnp.float32),
        ],
    )(degp, xpad)


def _tc_mid(p1, g1, dinv, W1p, b1, W2, b2, n_pad):
    grid = n_pad // BN
    H = W2.shape[0]

    def body(p_ref, g1_ref, dinv_ref, w1_ref, b1_ref, w2_ref, *out_refs):
        dinv = dinv_ref[...]
        ax = (p_ref[0] + p_ref[1] + g1_ref[...]) * dinv
        h1 = jnp.maximum(
            jnp.dot(ax, w1_ref[...], preferred_element_type=jnp.float32)
            + b1_ref[...], 0.0)
        g2 = jnp.dot(h1, w2_ref[...], preferred_element_type=jnp.float32) * dinv
        for f in range(4):
            out_refs[f][...] = g2[:, f * L:(f + 1) * L]

    return pl.pallas_call(
        body,
        grid=(grid,),
        in_specs=[
            pl.BlockSpec((NC, BN, L), lambda i: (0, i, 0)),
            pl.BlockSpec((BN, L), lambda i: (i, 0)),
            pl.BlockSpec((BN, 1), lambda i: (i, 0)),
            pl.BlockSpec((L, H), lambda i: (0, 0)),
            pl.BlockSpec((1, H), lambda i: (0, 0)),
            pl.BlockSpec((H, H), lambda i: (0, 0)),
        ],
        out_specs=[pl.BlockSpec((BN, L), lambda i: (i, 0)) for _ in range(4)],
        out_shape=[jax.ShapeDtypeStruct((n_pad, L), jnp.float32)
                   for _ in range(4)],
    )(p1, g1, dinv, W1p, b1.reshape(1, H), W2)


def _tc_final(agg, g2s, dinv, b2, Wl, bl, n_pad):
    grid = n_pad // BN
    H = Wl.shape[0]
    OUT = Wl.shape[1]

    def body(agg_ref, g0, g1, g2, g3, dinv_ref, b2_ref, wl_ref, bl_ref, out_ref):
        dinv = dinv_ref[...]
        gs = (g0, g1, g2, g3)
        parts = [agg_ref[f, 0] + agg_ref[f, 1] + gs[f][...] for f in range(4)]
        pre = jnp.concatenate(parts, axis=1) * dinv + b2_ref[...]
        h2 = jnp.maximum(pre, 0.0)
        out_ref[...] = (jnp.dot(h2, wl_ref[...],
                                preferred_element_type=jnp.float32)
                        + bl_ref[...])

    return pl.pallas_call(
        body,
        grid=(grid,),
        in_specs=[
            pl.BlockSpec((4, NC, BN, L), lambda i: (0, 0, i, 0)),
            pl.BlockSpec((BN, L), lambda i: (i, 0)),
            pl.BlockSpec((BN, L), lambda i: (i, 0)),
            pl.BlockSpec((BN, L), lambda i: (i, 0)),
            pl.BlockSpec((BN, L), lambda i: (i, 0)),
            pl.BlockSpec((BN, 1), lambda i: (i, 0)),
            pl.BlockSpec((1, H), lambda i: (0, 0)),
            pl.BlockSpec((H, OUT), lambda i: (0, 0)),
            pl.BlockSpec((1, OUT), lambda i: (0, 0)),
        ],
        out_specs=pl.BlockSpec((BN, OUT), lambda i: (i, 0)),
        out_shape=jax.ShapeDtypeStruct((n_pad, OUT), jnp.float32),
    )(agg, *g2s, dinv, b2.reshape(1, H), Wl, bl.reshape(1, OUT))


def kernel(x, edge_index, W1, b1, W2, b2, Wl, bl):
    N, IN = x.shape
    E = edge_index.shape[1]

    n_pad = -(-(N + 1) // BN) * BN
    ew = -(-E // (NW * CHUNK)) * CHUNK
    Ep = NW * ew
    src = jnp.concatenate(
        [edge_index[0], jnp.zeros((Ep - E,), jnp.int32)]).reshape(Ep // B, B)
    dst = jnp.concatenate(
        [edge_index[1], jnp.full((Ep - E,), N, jnp.int32)]).reshape(Ep // B, B)

    xpad = jnp.zeros((n_pad, L), jnp.float32).at[:N, :IN].set(x)
    W1p = jnp.pad(W1, ((0, L - IN), (0, 0)))

    degp = _sc_degree_pass(dst, n_pad)
    dinv, g1 = _tc_prep(degp, xpad, n_pad)
    p1 = _sc_scatter_pass(src, dst, [g1], n_pad)[0]
    g2s = _tc_mid(p1, g1, dinv, W1p, b1, W2, b2, n_pad)
    agg = _sc_scatter_pass(src, dst, list(g2s), n_pad)
    out = _tc_final(agg, g2s, dinv, b2, Wl, bl, n_pad)
    return out[:N]


# speedup vs baseline: 29.4792x; 1.6239x over previous
import functools

import jax
import jax.numpy as jnp
from jax import lax
from jax.experimental import pallas as pl
from jax.experimental.pallas import tpu as pltpu
from jax.experimental.pallas import tpu_sc as plsc

NC = 2
NS = 16
NW = NC * NS
L = 16
B = 128
CH = 8
CHUNK = CH * B
BN = 1024


def _sc_scatter_pass(src2d, dst2d, tables, n_pad):
    nt = len(tables)
    rows_w = src2d.shape[0] // NW
    n_chunks = rows_w // CH
    tile_rows = n_pad // NS
    mesh = plsc.VectorSubcoreMesh(core_axis_name="c", subcore_axis_name="s")

    @functools.partial(
        pl.kernel,
        out_type=jax.ShapeDtypeStruct((nt, NC, n_pad, L), jnp.float32),
        mesh=mesh,
        scratch_types=[
            pltpu.VMEM((CH, B), jnp.int32),
            pltpu.VMEM((CH, B), jnp.int32),
            pltpu.VMEM((CHUNK, L), jnp.float32),
            pltpu.VMEM_SHARED((n_pad, L), jnp.float32),
            pltpu.VMEM((64, L), jnp.float32),
            pltpu.SemaphoreType.DMA,
        ],
        compiler_params=pltpu.CompilerParams(use_tc_tiling_on_sc=False),
    )
    def k(src_hbm, dst_hbm, *rest):
        tabs = rest[:nt]
        out_hbm, src_v, dst_v, rows_v, acc, zbuf, sem = rest[nt:]
        c = lax.axis_index("c")
        s = lax.axis_index("s")
        wid = s * NC + c
        base_row = wid * rows_w
        t0 = s * tile_rows

        zeros16 = jnp.zeros((L,), jnp.float32)
        for r in range(64):
            zbuf[r, :] = zeros16

        for f in range(nt):
            def zero_body(i, _):
                pltpu.sync_copy(zbuf, acc.at[pl.ds(t0 + i * 64, 64)])
                return 0
            lax.fori_loop(0, tile_rows // 64, zero_body, 0)
            plsc.subcore_barrier()

            def body(i, _):
                r0 = base_row + i * CH
                pltpu.sync_copy(dst_hbm.at[pl.ds(r0, CH)], dst_v)
                pltpu.sync_copy(src_hbm.at[pl.ds(r0, CH)], src_v)
                cps = []
                for j in range(CH):
                    cps.append(pltpu.async_copy(
                        tabs[f].at[src_v.at[j]],
                        rows_v.at[pl.ds(j * B, B)], sem))
                for cp in cps:
                    cp.wait()
                for j in range(CH):
                    pltpu.sync_copy(rows_v.at[pl.ds(j * B, B)],
                                    acc.at[dst_v.at[j]], add=True)
                return 0
            lax.fori_loop(0, n_chunks, body, 0)
            plsc.subcore_barrier()

            pltpu.sync_copy(acc.at[pl.ds(t0, tile_rows)],
                            out_hbm.at[f, c, pl.ds(t0, tile_rows)])

    return k(src2d, dst2d, *tables)


def _sc_degree_pass(dst2d, n_pad):
    rows_w = dst2d.shape[0] // NW
    n_chunks = rows_w // CH
    tile_rows = n_pad // NS
    mesh = plsc.VectorSubcoreMesh(core_axis_name="c", subcore_axis_name="s")

    @functools.partial(
        pl.kernel,
        out_type=jax.ShapeDtypeStruct((NC, n_pad, L), jnp.float32),
        mesh=mesh,
        scratch_types=[
            pltpu.VMEM((CH, B), jnp.int32),
            pltpu.VMEM((B, L), jnp.float32),
            pltpu.VMEM_SHARED((n_pad, L), jnp.float32),
            pltpu.VMEM((64, L), jnp.float32),
        ],
        compiler_params=pltpu.CompilerParams(use_tc_tiling_on_sc=False),
    )
    def k(dst_hbm, out_hbm, dst_v, ones_v, acc, zbuf):
        c = lax.axis_index("c")
        s = lax.axis_index("s")
        wid = s * NC + c
        base_row = wid * rows_w
        t0 = s * tile_rows

        zeros16 = jnp.zeros((L,), jnp.float32)
        ones16 = jnp.ones((L,), jnp.float32)
        for r in range(64):
            zbuf[r, :] = zeros16
        for r in range(B):
            ones_v[r, :] = ones16

        def zero_body(i, _):
            pltpu.sync_copy(zbuf, acc.at[pl.ds(t0 + i * 64, 64)])
            return 0
        lax.fori_loop(0, tile_rows // 64, zero_body, 0)
        plsc.subcore_barrier()

        def body(i, _):
            r0 = base_row + i * CH
            pltpu.sync_copy(dst_hbm.at[pl.ds(r0, CH)], dst_v)
            for j in range(CH):
                pltpu.sync_copy(ones_v, acc.at[dst_v.at[j]], add=True)
            return 0
        lax.fori_loop(0, n_chunks, body, 0)
        plsc.subcore_barrier()

        pltpu.sync_copy(acc.at[pl.ds(t0, tile_rows)],
                        out_hbm.at[c, pl.ds(t0, tile_rows)])

    return k(dst2d)


BN8 = 448


def _pack_weights(W1p, b1, W2, b2, Wl):
    H = W2.shape[0]
    eye8 = jnp.eye(8, dtype=jnp.float32)
    f_, k_, t_ = jnp.meshgrid(jnp.arange(4), jnp.arange(8), jnp.arange(L),
                              indexing="ij")
    perm = (k_ * H + f_ * L + t_).reshape(-1)
    W1q = jnp.einsum("ab,ij->aibj", eye8, W1p).reshape(8 * L, 8 * H)[:, perm]
    W2q = jnp.einsum("ab,ij->aibj", eye8, W2).reshape(8 * H, 8 * H)[perm][:, perm]
    b1q = jnp.concatenate(
        [jnp.tile(b1[f * L:(f + 1) * L], 8) for f in range(4)]).reshape(1, 8 * H)
    b2q = jnp.stack([jnp.tile(b2[f * L:(f + 1) * L], 8) for f in range(4)])
    wlq = jnp.stack([jnp.tile(Wl[f * L:(f + 1) * L, 0], 8) for f in range(4)])
    return W1q, b1q, W2q, b2q, wlq


def _tc_prep(degq, xq, n8):
    def body(deg_ref, x_ref, dinv_ref, g1_ref):
        deg = 1.0 + deg_ref[0] + deg_ref[1]
        dinv = lax.rsqrt(deg)
        dinv_ref[...] = dinv
        g1_ref[...] = x_ref[...] * dinv

    return pl.pallas_call(
        body,
        grid=(n8 // BN8,),
        in_specs=[
            pl.BlockSpec((NC, BN8, 128), lambda i: (0, i, 0)),
            pl.BlockSpec((BN8, 128), lambda i: (i, 0)),
        ],
        out_specs=[
            pl.BlockSpec((BN8, 128), lambda i: (i, 0)),
            pl.BlockSpec((BN8, 128), lambda i: (i, 0)),
        ],
        out_shape=[
            jax.ShapeDtypeStruct((n8, 128), jnp.float32),
            jax.ShapeDtypeStruct((n8, 128), jnp.float32),
        ],
    )(degq, xq)


def _tc_mid(p1q, g1q, dinvq, W1q, b1q, W2q, n8):
    def body(p_ref, g1_ref, dinv_ref, w1_ref, b1_ref, w2_ref, *out_refs):
        dinv = dinv_ref[...]
        ax = (p_ref[0] + p_ref[1] + g1_ref[...]) * dinv
        h1 = jnp.maximum(
            jnp.dot(ax, w1_ref[...], preferred_element_type=jnp.float32)
            + b1_ref[...], 0.0)
        g2 = jnp.dot(h1, w2_ref[...], preferred_element_type=jnp.float32)
        for f in range(4):
            out_refs[f][...] = g2[:, f * 128:(f + 1) * 128] * dinv

    return pl.pallas_call(
        body,
        grid=(n8 // BN8,),
        in_specs=[
            pl.BlockSpec((NC, BN8, 128), lambda i: (0, i, 0)),
            pl.BlockSpec((BN8, 128), lambda i: (i, 0)),
            pl.BlockSpec((BN8, 128), lambda i: (i, 0)),
            pl.BlockSpec((128, 512), lambda i: (0, 0)),
            pl.BlockSpec((1, 512), lambda i: (0, 0)),
            pl.BlockSpec((512, 512), lambda i: (0, 0)),
        ],
        out_specs=[pl.BlockSpec((BN8, 128), lambda i: (i, 0))
                   for _ in range(4)],
        out_shape=[jax.ShapeDtypeStruct((n8, 128), jnp.float32)
                   for _ in range(4)],
    )(p1q, g1q, dinvq, W1q, b1q, W2q)


def _tc_final(aggq, g2qs, dinvq, b2q, wlq, bl, n8):
    M = jnp.repeat(jnp.eye(8, dtype=jnp.float32), L, axis=0)

    def body(agg_ref, g0, g1, g2, g3, dinv_ref, b2_ref, wl_ref, m_ref,
             bl_ref, out_ref):
        dinv = dinv_ref[...]
        gs = (g0, g1, g2, g3)
        s = jnp.zeros_like(dinv)
        for f in range(4):
            u = (agg_ref[f, 0] + agg_ref[f, 1] + gs[f][...]) * dinv \
                + b2_ref[f, 0]
            s = s + jnp.maximum(u, 0.0) * wl_ref[f, 0]
        out_ref[...] = (jnp.dot(s, m_ref[...],
                                preferred_element_type=jnp.float32)
                        + bl_ref[0, 0])

    return pl.pallas_call(
        body,
        grid=(n8 // BN8,),
        in_specs=[
            pl.BlockSpec((4, NC, BN8, 128), lambda i: (0, 0, i, 0)),
            pl.BlockSpec((BN8, 128), lambda i: (i, 0)),
            pl.BlockSpec((BN8, 128), lambda i: (i, 0)),
            pl.BlockSpec((BN8, 128), lambda i: (i, 0)),
            pl.BlockSpec((BN8, 128), lambda i: (i, 0)),
            pl.BlockSpec((BN8, 128), lambda i: (i, 0)),
            pl.BlockSpec((4, 1, 128), lambda i: (0, 0, 0)),
            pl.BlockSpec((4, 1, 128), lambda i: (0, 0, 0)),
            pl.BlockSpec((128, 8), lambda i: (0, 0)),
            pl.BlockSpec((1, 1), lambda i: (0, 0)),
        ],
        out_specs=pl.BlockSpec((BN8, 8), lambda i: (i, 0)),
        out_shape=jax.ShapeDtypeStruct((n8, 8), jnp.float32),
    )(aggq, *g2qs, dinvq, b2q.reshape(4, 1, 128), wlq.reshape(4, 1, 128),
      M, bl.reshape(1, 1))


def kernel(x, edge_index, W1, b1, W2, b2, Wl, bl):
    N, IN = x.shape
    E = edge_index.shape[1]

    n_pad = -(-(N + 1) // BN) * BN
    n8 = n_pad // 8
    ew = -(-E // (NW * CHUNK)) * CHUNK
    Ep = NW * ew
    src = jnp.concatenate(
        [edge_index[0], jnp.zeros((Ep - E,), jnp.int32)]).reshape(Ep // B, B)
    dst = jnp.concatenate(
        [edge_index[1], jnp.full((Ep - E,), N, jnp.int32)]).reshape(Ep // B, B)

    xq = jnp.zeros((n_pad, L), jnp.float32).at[:N, :IN].set(x).reshape(n8, 128)
    W1p = jnp.pad(W1, ((0, L - IN), (0, 0)))
    W1q, b1q, W2q, b2q, wlq = _pack_weights(W1p, b1, W2, b2, Wl)

    degp = _sc_degree_pass(dst, n_pad)
    dinvq, g1q = _tc_prep(degp.reshape(NC, n8, 128), xq, n8)
    g1 = g1q.reshape(n_pad, L)
    p1 = _sc_scatter_pass(src, dst, [g1], n_pad)[0]
    g2qs = _tc_mid(p1.reshape(NC, n8, 128), g1q, dinvq, W1q, b1q, W2q, n8)
    g2s = [g.reshape(n_pad, L) for g in g2qs]
    agg = _sc_scatter_pass(src, dst, g2s, n_pad)
    out = _tc_final(agg.reshape(4, NC, n8, 128), g2qs, dinvq, b2q, wlq, bl, n8)
    return out.reshape(n_pad, 1)[:N]


# speedup vs baseline: 38.4415x; 1.3040x over previous
import functools

import jax
import jax.numpy as jnp
from jax import lax
from jax.experimental import pallas as pl
from jax.experimental.pallas import tpu as pltpu
from jax.experimental.pallas import tpu_sc as plsc

NC = 2
NS = 16
NW = NC * NS
L = 16
B = 128
CH = 4
CHUNK = CH * B
BN = 1024


def _sc_scatter_pass(src2d, dst2d, tables, n_pad):
    nt = len(tables)
    rows_w = src2d.shape[0] // NW
    n_pairs = rows_w // (2 * CH)
    tile_rows = n_pad // NS
    mesh = plsc.VectorSubcoreMesh(core_axis_name="c", subcore_axis_name="s")

    @functools.partial(
        pl.kernel,
        out_type=jax.ShapeDtypeStruct((nt, NC, n_pad, L), jnp.float32),
        mesh=mesh,
        scratch_types=[
            pltpu.VMEM((2, CH, B), jnp.int32),
            pltpu.VMEM((2, CH, B), jnp.int32),
            pltpu.VMEM((2, CHUNK, L), jnp.float32),
            pltpu.VMEM_SHARED((n_pad, L), jnp.float32),
            pltpu.VMEM((64, L), jnp.float32),
            pltpu.SemaphoreType.DMA,
            pltpu.SemaphoreType.DMA,
            pltpu.SemaphoreType.DMA,
            pltpu.SemaphoreType.DMA,
            pltpu.SemaphoreType.DMA,
        ],
        compiler_params=pltpu.CompilerParams(use_tc_tiling_on_sc=False),
    )
    def k(src_hbm, dst_hbm, *rest):
        tabs = rest[:nt]
        (out_hbm, src_v, dst_v, rows_v, acc, zbuf,
         isem, gsem_a, gsem_b, ssem_a, ssem_b) = rest[nt:]
        c = lax.axis_index("c")
        s = lax.axis_index("s")
        wid = s * NC + c
        base_row = wid * rows_w
        t0 = s * tile_rows

        zeros16 = jnp.zeros((L,), jnp.float32)
        for r in range(64):
            zbuf[r, :] = zeros16

        for f in range(nt):
            def zero_body(i, _):
                pltpu.sync_copy(zbuf, acc.at[pl.ds(t0 + i * 64, 64)])
                return 0
            lax.fori_loop(0, tile_rows // 64, zero_body, 0)
            plsc.subcore_barrier()

            def drain_scatters(buf, sem):
                pltpu.make_async_copy(
                    tabs[f].at[pl.ds(0, CHUNK)], rows_v.at[buf], sem).wait()

            def stage_in(g2, buf, gsem, ssem, first):
                @pl.when(jnp.logical_not(first))
                def _():
                    drain_scatters(buf, ssem)
                r0 = base_row + g2 * CH
                cps = [pltpu.async_copy(src_hbm.at[pl.ds(r0, CH)],
                                        src_v.at[buf], isem),
                       pltpu.async_copy(dst_hbm.at[pl.ds(r0, CH)],
                                        dst_v.at[buf], isem)]
                for cp in cps:
                    cp.wait()
                return [pltpu.async_copy(tabs[f].at[src_v.at[buf, j]],
                                         rows_v.at[buf, pl.ds(j * B, B)], gsem)
                        for j in range(CH)]

            def scatter_out(buf, gathers, ssem):
                for cp in gathers:
                    cp.wait()
                for j in range(CH):
                    pltpu.async_copy(rows_v.at[buf, pl.ds(j * B, B)],
                                     acc.at[dst_v.at[buf, j]], ssem, add=True)

            def pair_body(g, _):
                first = g == 0
                ga = stage_in(2 * g, 0, gsem_a, ssem_a, first)
                gb = stage_in(2 * g + 1, 1, gsem_b, ssem_b, first)
                scatter_out(0, ga, ssem_a)
                scatter_out(1, gb, ssem_b)
                return 0
            lax.fori_loop(0, n_pairs, pair_body, 0)
            drain_scatters(0, ssem_a)
            drain_scatters(1, ssem_b)
            plsc.subcore_barrier()

            pltpu.sync_copy(acc.at[pl.ds(t0, tile_rows)],
                            out_hbm.at[f, c, pl.ds(t0, tile_rows)])

    return k(src2d, dst2d, *tables)


def _sc_degree_pass(dst2d, n_pad):
    rows_w = dst2d.shape[0] // NW
    n_chunks = rows_w // CH
    tile_rows = n_pad // NS
    mesh = plsc.VectorSubcoreMesh(core_axis_name="c", subcore_axis_name="s")

    @functools.partial(
        pl.kernel,
        out_type=jax.ShapeDtypeStruct((NC, n_pad, L), jnp.float32),
        mesh=mesh,
        scratch_types=[
            pltpu.VMEM((CH, B), jnp.int32),
            pltpu.VMEM((B, L), jnp.float32),
            pltpu.VMEM_SHARED((n_pad, L), jnp.float32),
            pltpu.VMEM((64, L), jnp.float32),
        ],
        compiler_params=pltpu.CompilerParams(use_tc_tiling_on_sc=False),
    )
    def k(dst_hbm, out_hbm, dst_v, ones_v, acc, zbuf):
        c = lax.axis_index("c")
        s = lax.axis_index("s")
        wid = s * NC + c
        base_row = wid * rows_w
        t0 = s * tile_rows

        zeros16 = jnp.zeros((L,), jnp.float32)
        ones16 = jnp.ones((L,), jnp.float32)
        for r in range(64):
            zbuf[r, :] = zeros16
        for r in range(B):
            ones_v[r, :] = ones16

        def zero_body(i, _):
            pltpu.sync_copy(zbuf, acc.at[pl.ds(t0 + i * 64, 64)])
            return 0
        lax.fori_loop(0, tile_rows // 64, zero_body, 0)
        plsc.subcore_barrier()

        def body(i, _):
            r0 = base_row + i * CH
            pltpu.sync_copy(dst_hbm.at[pl.ds(r0, CH)], dst_v)
            for j in range(CH):
                pltpu.sync_copy(ones_v, acc.at[dst_v.at[j]], add=True)
            return 0
        lax.fori_loop(0, n_chunks, body, 0)
        plsc.subcore_barrier()

        pltpu.sync_copy(acc.at[pl.ds(t0, tile_rows)],
                        out_hbm.at[c, pl.ds(t0, tile_rows)])

    return k(dst2d)


BN8 = 448


def _pack_weights(W1p, b1, W2, b2, Wl):
    H = W2.shape[0]
    eye8 = jnp.eye(8, dtype=jnp.float32)
    f_, k_, t_ = jnp.meshgrid(jnp.arange(4), jnp.arange(8), jnp.arange(L),
                              indexing="ij")
    perm = (k_ * H + f_ * L + t_).reshape(-1)
    W1q = jnp.einsum("ab,ij->aibj", eye8, W1p).reshape(8 * L, 8 * H)[:, perm]
    W2q = jnp.einsum("ab,ij->aibj", eye8, W2).reshape(8 * H, 8 * H)[perm][:, perm]
    b1q = jnp.concatenate(
        [jnp.tile(b1[f * L:(f + 1) * L], 8) for f in range(4)]).reshape(1, 8 * H)
    b2q = jnp.stack([jnp.tile(b2[f * L:(f + 1) * L], 8) for f in range(4)])
    wlq = jnp.stack([jnp.tile(Wl[f * L:(f + 1) * L, 0], 8) for f in range(4)])
    return W1q, b1q, W2q, b2q, wlq


def _tc_prep(degq, xq, n8):
    def body(deg_ref, x_ref, dinv_ref, g1_ref):
        deg = 1.0 + deg_ref[0] + deg_ref[1]
        dinv = lax.rsqrt(deg)
        dinv_ref[...] = dinv
        g1_ref[...] = x_ref[...] * dinv

    return pl.pallas_call(
        body,
        grid=(n8 // BN8,),
        in_specs=[
            pl.BlockSpec((NC, BN8, 128), lambda i: (0, i, 0)),
            pl.BlockSpec((BN8, 128), lambda i: (i, 0)),
        ],
        out_specs=[
            pl.BlockSpec((BN8, 128), lambda i: (i, 0)),
            pl.BlockSpec((BN8, 128), lambda i: (i, 0)),
        ],
        out_shape=[
            jax.ShapeDtypeStruct((n8, 128), jnp.float32),
            jax.ShapeDtypeStruct((n8, 128), jnp.float32),
        ],
    )(degq, xq)


def _tc_mid(p1q, g1q, dinvq, W1q, b1q, W2q, n8):
    def body(p_ref, g1_ref, dinv_ref, w1_ref, b1_ref, w2_ref, *out_refs):
        dinv = dinv_ref[...]
        ax = (p_ref[0] + p_ref[1] + g1_ref[...]) * dinv
        h1 = jnp.maximum(
            jnp.dot(ax, w1_ref[...], preferred_element_type=jnp.float32)
            + b1_ref[...], 0.0)
        g2 = jnp.dot(h1, w2_ref[...], preferred_element_type=jnp.float32)
        for f in range(4):
            out_refs[f][...] = g2[:, f * 128:(f + 1) * 128] * dinv

    return pl.pallas_call(
        body,
        grid=(n8 // BN8,),
        in_specs=[
            pl.BlockSpec((NC, BN8, 128), lambda i: (0, i, 0)),
            pl.BlockSpec((BN8, 128), lambda i: (i, 0)),
            pl.BlockSpec((BN8, 128), lambda i: (i, 0)),
            pl.BlockSpec((128, 512), lambda i: (0, 0)),
            pl.BlockSpec((1, 512), lambda i: (0, 0)),
            pl.BlockSpec((512, 512), lambda i: (0, 0)),
        ],
        out_specs=[pl.BlockSpec((BN8, 128), lambda i: (i, 0))
                   for _ in range(4)],
        out_shape=[jax.ShapeDtypeStruct((n8, 128), jnp.float32)
                   for _ in range(4)],
    )(p1q, g1q, dinvq, W1q, b1q, W2q)


def _tc_final(aggq, g2qs, dinvq, b2q, wlq, bl, n8):
    M = jnp.repeat(jnp.eye(8, dtype=jnp.float32), L, axis=0)

    def body(agg_ref, g0, g1, g2, g3, dinv_ref, b2_ref, wl_ref, m_ref,
             bl_ref, out_ref):
        dinv = dinv_ref[...]
        gs = (g0, g1, g2, g3)
        s = jnp.zeros_like(dinv)
        for f in range(4):
            u = (agg_ref[f, 0] + agg_ref[f, 1] + gs[f][...]) * dinv \
                + b2_ref[f, 0]
            s = s + jnp.maximum(u, 0.0) * wl_ref[f, 0]
        out_ref[...] = (jnp.dot(s, m_ref[...],
                                preferred_element_type=jnp.float32)
                        + bl_ref[0, 0])

    return pl.pallas_call(
        body,
        grid=(n8 // BN8,),
        in_specs=[
            pl.BlockSpec((4, NC, BN8, 128), lambda i: (0, 0, i, 0)),
            pl.BlockSpec((BN8, 128), lambda i: (i, 0)),
            pl.BlockSpec((BN8, 128), lambda i: (i, 0)),
            pl.BlockSpec((BN8, 128), lambda i: (i, 0)),
            pl.BlockSpec((BN8, 128), lambda i: (i, 0)),
            pl.BlockSpec((BN8, 128), lambda i: (i, 0)),
            pl.BlockSpec((4, 1, 128), lambda i: (0, 0, 0)),
            pl.BlockSpec((4, 1, 128), lambda i: (0, 0, 0)),
            pl.BlockSpec((128, 8), lambda i: (0, 0)),
            pl.BlockSpec((1, 1), lambda i: (0, 0)),
        ],
        out_specs=pl.BlockSpec((BN8, 8), lambda i: (i, 0)),
        out_shape=jax.ShapeDtypeStruct((n8, 8), jnp.float32),
    )(aggq, *g2qs, dinvq, b2q.reshape(4, 1, 128), wlq.reshape(4, 1, 128),
      M, bl.reshape(1, 1))


def kernel(x, edge_index, W1, b1, W2, b2, Wl, bl):
    N, IN = x.shape
    E = edge_index.shape[1]

    n_pad = -(-(N + 1) // BN) * BN
    n8 = n_pad // 8
    ew = -(-E // (NW * 2 * CHUNK)) * 2 * CHUNK
    Ep = NW * ew
    src = jnp.concatenate(
        [edge_index[0], jnp.zeros((Ep - E,), jnp.int32)]).reshape(Ep // B, B)
    dst = jnp.concatenate(
        [edge_index[1], jnp.full((Ep - E,), N, jnp.int32)]).reshape(Ep // B, B)

    xq = jnp.zeros((n_pad, L), jnp.float32).at[:N, :IN].set(x).reshape(n8, 128)
    W1p = jnp.pad(W1, ((0, L - IN), (0, 0)))
    W1q, b1q, W2q, b2q, wlq = _pack_weights(W1p, b1, W2, b2, Wl)

    degp = _sc_degree_pass(dst, n_pad)
    dinvq, g1q = _tc_prep(degp.reshape(NC, n8, 128), xq, n8)
    g1 = g1q.reshape(n_pad, L)
    p1 = _sc_scatter_pass(src, dst, [g1], n_pad)[0]
    g2qs = _tc_mid(p1.reshape(NC, n8, 128), g1q, dinvq, W1q, b1q, W2q, n8)
    g2s = [g.reshape(n_pad, L) for g in g2qs]
    agg = _sc_scatter_pass(src, dst, g2s, n_pad)
    out = _tc_final(agg.reshape(4, NC, n8, 128), g2qs, dinvq, b2q, wlq, bl, n8)
    return out.reshape(n_pad, 1)[:N]


# speedup vs baseline: 40.8590x; 1.0629x over previous
import functools

import jax
import jax.numpy as jnp
from jax import lax
from jax.experimental import pallas as pl
from jax.experimental.pallas import tpu as pltpu
from jax.experimental.pallas import tpu_sc as plsc

NC = 2
NS = 16
NW = NC * NS
L = 16
B = 128
CH = 4
CHUNK = CH * B
BN = 1024


def _sc_scatter_pass(edges, tables, n_pad):
    nt = len(tables)
    rows_w = edges.shape[0] // 2 // NW
    n_pairs = rows_w // (2 * CH)
    tile_rows = n_pad // NS
    mesh = plsc.VectorSubcoreMesh(core_axis_name="c", subcore_axis_name="s")

    @functools.partial(
        pl.kernel,
        out_type=jax.ShapeDtypeStruct((nt, NC, n_pad, L), jnp.float32),
        mesh=mesh,
        scratch_types=[
            pltpu.VMEM((2, 2 * CH, B), jnp.int32),
            pltpu.VMEM((2, CHUNK, L), jnp.float32),
            pltpu.VMEM_SHARED((n_pad, L), jnp.float32),
            pltpu.VMEM((64, L), jnp.float32),
            pltpu.SemaphoreType.DMA,
            pltpu.SemaphoreType.DMA,
            pltpu.SemaphoreType.DMA,
            pltpu.SemaphoreType.DMA,
            pltpu.SemaphoreType.DMA,
        ],
        compiler_params=pltpu.CompilerParams(use_tc_tiling_on_sc=False),
    )
    def k(edges_hbm, *rest):
        tabs = rest[:nt]
        (out_hbm, idx_v, rows_v, acc, zbuf,
         isem, gsem_a, gsem_b, ssem_a, ssem_b) = rest[nt:]
        c = lax.axis_index("c")
        s = lax.axis_index("s")
        wid = s * NC + c
        base_row = wid * rows_w
        t0 = s * tile_rows

        zeros16 = jnp.zeros((L,), jnp.float32)
        for r in range(64):
            zbuf[r, :] = zeros16

        for f in range(nt):
            def zero_body(i, _):
                pltpu.sync_copy(zbuf, acc.at[pl.ds(t0 + i * 64, 64)])
                return 0
            lax.fori_loop(0, tile_rows // 64, zero_body, 0)
            plsc.subcore_barrier()

            def drain_scatters(buf, sem):
                pltpu.make_async_copy(
                    tabs[f].at[pl.ds(0, CHUNK)], rows_v.at[buf], sem).wait()

            def stage_in(g2, buf, gsem, ssem, first):
                @pl.when(jnp.logical_not(first))
                def _():
                    drain_scatters(buf, ssem)
                r0 = 2 * (base_row + g2 * CH)
                pltpu.async_copy(edges_hbm.at[pl.ds(r0, 2 * CH)],
                                 idx_v.at[buf], isem).wait()
                return [pltpu.async_copy(tabs[f].at[idx_v.at[buf, 2 * j]],
                                         rows_v.at[buf, pl.ds(j * B, B)], gsem)
                        for j in range(CH)]

            def scatter_out(buf, gathers, ssem):
                for cp in gathers:
                    cp.wait()
                for j in range(CH):
                    pltpu.async_copy(rows_v.at[buf, pl.ds(j * B, B)],
                                     acc.at[idx_v.at[buf, 2 * j + 1]], ssem,
                                     add=True)

            def pair_body(g, _):
                first = g == 0
                ga = stage_in(2 * g, 0, gsem_a, ssem_a, first)
                gb = stage_in(2 * g + 1, 1, gsem_b, ssem_b, first)
                scatter_out(0, ga, ssem_a)
                scatter_out(1, gb, ssem_b)
                return 0
            lax.fori_loop(0, n_pairs, pair_body, 0)
            drain_scatters(0, ssem_a)
            drain_scatters(1, ssem_b)
            plsc.subcore_barrier()

            pltpu.sync_copy(acc.at[pl.ds(t0, tile_rows)],
                            out_hbm.at[f, c, pl.ds(t0, tile_rows)])

    return k(edges, *tables)


CHD = 8


def _sc_degree_pass(edges, n_pad):
    rows_w = edges.shape[0] // 2 // NW
    n_chunks = rows_w // CHD
    tile_rows = n_pad // NS
    mesh = plsc.VectorSubcoreMesh(core_axis_name="c", subcore_axis_name="s")

    @functools.partial(
        pl.kernel,
        out_type=jax.ShapeDtypeStruct((NC, n_pad, L), jnp.float32),
        mesh=mesh,
        scratch_types=[
            pltpu.VMEM((2, 2 * CHD, B), jnp.int32),
            pltpu.VMEM((B, L), jnp.float32),
            pltpu.VMEM_SHARED((n_pad, L), jnp.float32),
            pltpu.VMEM((64, L), jnp.float32),
            pltpu.SemaphoreType.DMA,
            pltpu.SemaphoreType.DMA,
        ],
        compiler_params=pltpu.CompilerParams(use_tc_tiling_on_sc=False),
    )
    def k(edges_hbm, out_hbm, idx_v, ones_v, acc, zbuf, isem, ssem):
        c = lax.axis_index("c")
        s = lax.axis_index("s")
        wid = s * NC + c
        base_row = wid * rows_w
        t0 = s * tile_rows

        zeros16 = jnp.zeros((L,), jnp.float32)
        ones16 = jnp.ones((L,), jnp.float32)
        for r in range(64):
            zbuf[r, :] = zeros16
        for r in range(B):
            ones_v[r, :] = ones16

        def zero_body(i, _):
            pltpu.sync_copy(zbuf, acc.at[pl.ds(t0 + i * 64, 64)])
            return 0
        lax.fori_loop(0, tile_rows // 64, zero_body, 0)
        plsc.subcore_barrier()

        def fetch(i, buf):
            pltpu.async_copy(
                edges_hbm.at[pl.ds(2 * (base_row + i * CHD), 2 * CHD)],
                idx_v.at[buf], isem)

        def drain_scatters():
            for _j in range(CHD):
                pltpu.make_async_copy(
                    out_hbm.at[0, pl.ds(0, B)], ones_v, ssem).wait()

        fetch(0, 0)

        def body(i, _):
            buf = lax.rem(i, 2)
            pltpu.make_async_copy(
                edges_hbm.at[pl.ds(0, 2 * CHD)], idx_v.at[0], isem).wait()

            @pl.when(i > 0)
            def _():
                drain_scatters()

            @pl.when(i + 1 < n_chunks)
            def _():
                fetch(i + 1, 1 - buf)
            for b in range(2):
                @pl.when(buf == b)
                def _():
                    for j in range(CHD):
                        pltpu.async_copy(ones_v,
                                         acc.at[idx_v.at[b, 2 * j + 1]],
                                         ssem, add=True)
            return 0
        lax.fori_loop(0, n_chunks, body, 0)
        drain_scatters()
        plsc.subcore_barrier()

        pltpu.sync_copy(acc.at[pl.ds(t0, tile_rows)],
                        out_hbm.at[c, pl.ds(t0, tile_rows)])

    return k(edges)


BN8 = 448


def _pack_weights(W1p, b1, W2, b2, Wl):
    H = W2.shape[0]
    eye8 = jnp.eye(8, dtype=jnp.float32)
    f_, k_, t_ = jnp.meshgrid(jnp.arange(4), jnp.arange(8), jnp.arange(L),
                              indexing="ij")
    perm = (k_ * H + f_ * L + t_).reshape(-1)
    W1q = jnp.einsum("ab,ij->aibj", eye8, W1p).reshape(8 * L, 8 * H)[:, perm]
    W2q = jnp.einsum("ab,ij->aibj", eye8, W2).reshape(8 * H, 8 * H)[perm][:, perm]
    b1q = jnp.concatenate(
        [jnp.tile(b1[f * L:(f + 1) * L], 8) for f in range(4)]).reshape(1, 8 * H)
    b2q = jnp.stack([jnp.tile(b2[f * L:(f + 1) * L], 8) for f in range(4)])
    wlq = jnp.stack([jnp.tile(Wl[f * L:(f + 1) * L, 0], 8) for f in range(4)])
    return W1q, b1q, W2q, b2q, wlq


def _tc_prep(degq, xq, n8):
    def body(deg_ref, x_ref, dinv_ref, g1_ref):
        deg = 1.0 + deg_ref[0] + deg_ref[1]
        dinv = lax.rsqrt(deg)
        dinv_ref[...] = dinv
        g1_ref[...] = x_ref[...] * dinv

    return pl.pallas_call(
        body,
        grid=(n8 // BN8,),
        in_specs=[
            pl.BlockSpec((NC, BN8, 128), lambda i: (0, i, 0)),
            pl.BlockSpec((BN8, 128), lambda i: (i, 0)),
        ],
        out_specs=[
            pl.BlockSpec((BN8, 128), lambda i: (i, 0)),
            pl.BlockSpec((BN8, 128), lambda i: (i, 0)),
        ],
        out_shape=[
            jax.ShapeDtypeStruct((n8, 128), jnp.float32),
            jax.ShapeDtypeStruct((n8, 128), jnp.float32),
        ],
    )(degq, xq)


def _tc_mid(p1q, g1q, dinvq, W1q, b1q, W2q, n8):
    def body(p_ref, g1_ref, dinv_ref, w1_ref, b1_ref, w2_ref, *out_refs):
        dinv = dinv_ref[...]
        ax = (p_ref[0] + p_ref[1] + g1_ref[...]) * dinv
        h1 = jnp.maximum(
            jnp.dot(ax, w1_ref[...], preferred_element_type=jnp.float32)
            + b1_ref[...], 0.0)
        g2 = jnp.dot(h1, w2_ref[...], preferred_element_type=jnp.float32)
        for f in range(4):
            out_refs[f][...] = g2[:, f * 128:(f + 1) * 128] * dinv

    return pl.pallas_call(
        body,
        grid=(n8 // BN8,),
        in_specs=[
            pl.BlockSpec((NC, BN8, 128), lambda i: (0, i, 0)),
            pl.BlockSpec((BN8, 128), lambda i: (i, 0)),
            pl.BlockSpec((BN8, 128), lambda i: (i, 0)),
            pl.BlockSpec((128, 512), lambda i: (0, 0)),
            pl.BlockSpec((1, 512), lambda i: (0, 0)),
            pl.BlockSpec((512, 512), lambda i: (0, 0)),
        ],
        out_specs=[pl.BlockSpec((BN8, 128), lambda i: (i, 0))
                   for _ in range(4)],
        out_shape=[jax.ShapeDtypeStruct((n8, 128), jnp.float32)
                   for _ in range(4)],
    )(p1q, g1q, dinvq, W1q, b1q, W2q)


def _tc_final(aggq, g2qs, dinvq, b2q, wlq, bl, n8):
    M = jnp.repeat(jnp.eye(8, dtype=jnp.float32), L, axis=0)

    def body(agg_ref, g0, g1, g2, g3, dinv_ref, b2_ref, wl_ref, m_ref,
             bl_ref, out_ref):
        dinv = dinv_ref[...]
        gs = (g0, g1, g2, g3)
        s = jnp.zeros_like(dinv)
        for f in range(4):
            u = (agg_ref[f, 0] + agg_ref[f, 1] + gs[f][...]) * dinv \
                + b2_ref[f, 0]
            s = s + jnp.maximum(u, 0.0) * wl_ref[f, 0]
        out_ref[...] = (jnp.dot(s, m_ref[...],
                                preferred_element_type=jnp.float32)
                        + bl_ref[0, 0])

    return pl.pallas_call(
        body,
        grid=(n8 // BN8,),
        in_specs=[
            pl.BlockSpec((4, NC, BN8, 128), lambda i: (0, 0, i, 0)),
            pl.BlockSpec((BN8, 128), lambda i: (i, 0)),
            pl.BlockSpec((BN8, 128), lambda i: (i, 0)),
            pl.BlockSpec((BN8, 128), lambda i: (i, 0)),
            pl.BlockSpec((BN8, 128), lambda i: (i, 0)),
            pl.BlockSpec((BN8, 128), lambda i: (i, 0)),
            pl.BlockSpec((4, 1, 128), lambda i: (0, 0, 0)),
            pl.BlockSpec((4, 1, 128), lambda i: (0, 0, 0)),
            pl.BlockSpec((128, 8), lambda i: (0, 0)),
            pl.BlockSpec((1, 1), lambda i: (0, 0)),
        ],
        out_specs=pl.BlockSpec((BN8, 8), lambda i: (i, 0)),
        out_shape=jax.ShapeDtypeStruct((n8, 8), jnp.float32),
    )(aggq, *g2qs, dinvq, b2q.reshape(4, 1, 128), wlq.reshape(4, 1, 128),
      M, bl.reshape(1, 1))


def kernel(x, edge_index, W1, b1, W2, b2, Wl, bl):
    N, IN = x.shape
    E = edge_index.shape[1]

    n_pad = -(-(N + 1) // BN) * BN
    n8 = n_pad // 8
    ew = -(-E // (NW * 2 * CHUNK)) * 2 * CHUNK
    Ep = NW * ew
    e3t = jnp.transpose(edge_index.reshape(2, E // B, B), (1, 0, 2))
    pad_rows = (Ep - E) // B
    tail = jnp.concatenate(
        [jnp.zeros((pad_rows, 1, B), jnp.int32),
         jnp.full((pad_rows, 1, B), N, jnp.int32)], axis=1)
    edges = jnp.concatenate([e3t, tail], axis=0).reshape(-1, B)

    xq = jnp.zeros((n_pad, L), jnp.float32).at[:N, :IN].set(x).reshape(n8, 128)
    W1p = jnp.pad(W1, ((0, L - IN), (0, 0)))
    W1q, b1q, W2q, b2q, wlq = _pack_weights(W1p, b1, W2, b2, Wl)

    degp = _sc_degree_pass(edges, n_pad)
    dinvq, g1q = _tc_prep(degp.reshape(NC, n8, 128), xq, n8)
    g1 = g1q.reshape(n_pad, L)
    p1 = _sc_scatter_pass(edges, [g1], n_pad)[0]
    g2qs = _tc_mid(p1.reshape(NC, n8, 128), g1q, dinvq, W1q, b1q, W2q, n8)
    g2s = [g.reshape(n_pad, L) for g in g2qs]
    agg = _sc_scatter_pass(edges, g2s, n_pad)
    out = _tc_final(agg.reshape(4, NC, n8, 128), g2qs, dinvq, b2q, wlq, bl, n8)
    return out.reshape(n_pad, 1)[:N]


# speedup vs baseline: 45.0816x; 1.1033x over previous
import functools

import jax
import jax.numpy as jnp
from jax import lax
from jax.experimental import pallas as pl
from jax.experimental.pallas import tpu as pltpu
from jax.experimental.pallas import tpu_sc as plsc

NC = 2
NS = 16
NW = NC * NS
L = 16
B = 128
CH = 4
CHUNK = CH * B
BN = 1024


def _sc_scatter_pass(edges, tables, n_pad):
    nt = len(tables)
    rows_w = edges.shape[0] // 2 // NW
    n_pairs = rows_w // (2 * CH)
    tile_rows = n_pad // NS
    mesh = plsc.VectorSubcoreMesh(core_axis_name="c", subcore_axis_name="s")

    @functools.partial(
        pl.kernel,
        out_type=jax.ShapeDtypeStruct((nt, NC, n_pad, L), jnp.float32),
        mesh=mesh,
        scratch_types=[
            pltpu.VMEM((2, 4 * CH, B), jnp.int32),
            pltpu.VMEM((2, CHUNK, L), jnp.float32),
            pltpu.VMEM_SHARED((n_pad, L), jnp.float32),
            pltpu.VMEM((64, L), jnp.float32),
            pltpu.SemaphoreType.DMA,
            pltpu.SemaphoreType.DMA,
            pltpu.SemaphoreType.DMA,
            pltpu.SemaphoreType.DMA,
            pltpu.SemaphoreType.DMA,
        ],
        compiler_params=pltpu.CompilerParams(use_tc_tiling_on_sc=False),
    )
    def k(edges_hbm, *rest):
        tabs = rest[:nt]
        (out_hbm, idx_v, rows_v, acc, zbuf,
         isem, gsem_a, gsem_b, ssem_a, ssem_b) = rest[nt:]
        c = lax.axis_index("c")
        s = lax.axis_index("s")
        wid = s * NC + c
        base_row = wid * rows_w
        t0 = s * tile_rows

        zeros16 = jnp.zeros((L,), jnp.float32)
        for r in range(64):
            zbuf[r, :] = zeros16

        for f in range(nt):
            def zero_body(i, _):
                pltpu.sync_copy(zbuf, acc.at[pl.ds(t0 + i * 64, 64)])
                return 0
            lax.fori_loop(0, tile_rows // 64, zero_body, 0)
            plsc.subcore_barrier()

            def drain_scatters(buf, sem):
                pltpu.make_async_copy(
                    tabs[f].at[pl.ds(0, CHUNK)], rows_v.at[buf], sem).wait()

            def fetch_pair(g, slot):
                pltpu.async_copy(
                    edges_hbm.at[pl.ds(2 * (base_row + 2 * g * CH), 4 * CH)],
                    idx_v.at[slot], isem)

            def gather_in(slot, half, buf, gsem, ssem, first):
                @pl.when(jnp.logical_not(first))
                def _():
                    drain_scatters(buf, ssem)
                off = 2 * CH * half
                return [pltpu.async_copy(
                    tabs[f].at[idx_v.at[slot, off + 2 * j]],
                    rows_v.at[buf, pl.ds(j * B, B)], gsem)
                    for j in range(CH)]

            def scatter_out(slot, half, buf, gathers, ssem):
                off = 2 * CH * half
                for cp in gathers:
                    cp.wait()
                for j in range(CH):
                    pltpu.async_copy(rows_v.at[buf, pl.ds(j * B, B)],
                                     acc.at[idx_v.at[slot, off + 2 * j + 1]],
                                     ssem, add=True)

            fetch_pair(0, 0)

            def pair_body(g, _):
                first = g == 0
                slot = lax.rem(g, 2)
                pltpu.make_async_copy(
                    edges_hbm.at[pl.ds(0, 4 * CH)], idx_v.at[0], isem).wait()
                ga = gather_in(slot, 0, 0, gsem_a, ssem_a, first)
                gb = gather_in(slot, 1, 1, gsem_b, ssem_b, first)

                @pl.when(g + 1 < n_pairs)
                def _():
                    fetch_pair(g + 1, 1 - slot)
                scatter_out(slot, 0, 0, ga, ssem_a)
                scatter_out(slot, 1, 1, gb, ssem_b)
                return 0
            lax.fori_loop(0, n_pairs, pair_body, 0)
            drain_scatters(0, ssem_a)
            drain_scatters(1, ssem_b)
            plsc.subcore_barrier()

            pltpu.sync_copy(acc.at[pl.ds(t0, tile_rows)],
                            out_hbm.at[f, c, pl.ds(t0, tile_rows)])

    return k(edges, *tables)


CHD = 8


def _sc_degree_pass(edges, n_pad):
    rows_w = edges.shape[0] // 2 // NW
    n_chunks = rows_w // CHD
    tile_rows = n_pad // NS
    mesh = plsc.VectorSubcoreMesh(core_axis_name="c", subcore_axis_name="s")

    @functools.partial(
        pl.kernel,
        out_type=jax.ShapeDtypeStruct((NC, n_pad, L), jnp.float32),
        mesh=mesh,
        scratch_types=[
            pltpu.VMEM((2, 2 * CHD, B), jnp.int32),
            pltpu.VMEM((B, L), jnp.float32),
            pltpu.VMEM_SHARED((n_pad, L), jnp.float32),
            pltpu.VMEM((64, L), jnp.float32),
            pltpu.SemaphoreType.DMA,
            pltpu.SemaphoreType.DMA,
        ],
        compiler_params=pltpu.CompilerParams(use_tc_tiling_on_sc=False),
    )
    def k(edges_hbm, out_hbm, idx_v, ones_v, acc, zbuf, isem, ssem):
        c = lax.axis_index("c")
        s = lax.axis_index("s")
        wid = s * NC + c
        base_row = wid * rows_w
        t0 = s * tile_rows

        zeros16 = jnp.zeros((L,), jnp.float32)
        ones16 = jnp.ones((L,), jnp.float32)
        for r in range(64):
            zbuf[r, :] = zeros16
        for r in range(B):
            ones_v[r, :] = ones16

        def zero_body(i, _):
            pltpu.sync_copy(zbuf, acc.at[pl.ds(t0 + i * 64, 64)])
            return 0
        lax.fori_loop(0, tile_rows // 64, zero_body, 0)
        plsc.subcore_barrier()

        def fetch(i, buf):
            pltpu.async_copy(
                edges_hbm.at[pl.ds(2 * (base_row + i * CHD), 2 * CHD)],
                idx_v.at[buf], isem)

        def drain_scatters():
            for _j in range(CHD):
                pltpu.make_async_copy(
                    out_hbm.at[0, pl.ds(0, B)], ones_v, ssem).wait()

        fetch(0, 0)

        def body(i, _):
            buf = lax.rem(i, 2)
            pltpu.make_async_copy(
                edges_hbm.at[pl.ds(0, 2 * CHD)], idx_v.at[0], isem).wait()

            @pl.when(i > 0)
            def _():
                drain_scatters()

            @pl.when(i + 1 < n_chunks)
            def _():
                fetch(i + 1, 1 - buf)
            for b in range(2):
                @pl.when(buf == b)
                def _():
                    for j in range(CHD):
                        pltpu.async_copy(ones_v,
                                         acc.at[idx_v.at[b, 2 * j + 1]],
                                         ssem, add=True)
            return 0
        lax.fori_loop(0, n_chunks, body, 0)
        drain_scatters()
        plsc.subcore_barrier()

        pltpu.sync_copy(acc.at[pl.ds(t0, tile_rows)],
                        out_hbm.at[c, pl.ds(t0, tile_rows)])

    return k(edges)


BN8 = 448


def _pack_weights(W1p, b1, W2, b2, Wl):
    H = W2.shape[0]
    eye8 = jnp.eye(8, dtype=jnp.float32)
    f_, k_, t_ = jnp.meshgrid(jnp.arange(4), jnp.arange(8), jnp.arange(L),
                              indexing="ij")
    perm = (k_ * H + f_ * L + t_).reshape(-1)
    W1q = jnp.einsum("ab,ij->aibj", eye8, W1p).reshape(8 * L, 8 * H)[:, perm]
    W2q = jnp.einsum("ab,ij->aibj", eye8, W2).reshape(8 * H, 8 * H)[perm][:, perm]
    b1q = jnp.concatenate(
        [jnp.tile(b1[f * L:(f + 1) * L], 8) for f in range(4)]).reshape(1, 8 * H)
    b2q = jnp.stack([jnp.tile(b2[f * L:(f + 1) * L], 8) for f in range(4)])
    wlq = jnp.stack([jnp.tile(Wl[f * L:(f + 1) * L, 0], 8) for f in range(4)])
    return W1q, b1q, W2q, b2q, wlq


def _tc_prep(degq, xq, n8):
    def body(deg_ref, x_ref, dinv_ref, g1_ref):
        deg = 1.0 + deg_ref[0] + deg_ref[1]
        dinv = lax.rsqrt(deg)
        dinv_ref[...] = dinv
        g1_ref[...] = x_ref[...] * dinv

    return pl.pallas_call(
        body,
        grid=(n8 // BN8,),
        in_specs=[
            pl.BlockSpec((NC, BN8, 128), lambda i: (0, i, 0)),
            pl.BlockSpec((BN8, 128), lambda i: (i, 0)),
        ],
        out_specs=[
            pl.BlockSpec((BN8, 128), lambda i: (i, 0)),
            pl.BlockSpec((BN8, 128), lambda i: (i, 0)),
        ],
        out_shape=[
            jax.ShapeDtypeStruct((n8, 128), jnp.float32),
            jax.ShapeDtypeStruct((n8, 128), jnp.float32),
        ],
    )(degq, xq)


def _tc_mid(p1q, g1q, dinvq, W1q, b1q, W2q, n8):
    def body(p_ref, g1_ref, dinv_ref, w1_ref, b1_ref, w2_ref, *out_refs):
        dinv = dinv_ref[...]
        ax = (p_ref[0] + p_ref[1] + g1_ref[...]) * dinv
        h1 = jnp.maximum(
            jnp.dot(ax, w1_ref[...], preferred_element_type=jnp.float32)
            + b1_ref[...], 0.0)
        g2 = jnp.dot(h1, w2_ref[...], preferred_element_type=jnp.float32)
        for f in range(4):
            out_refs[f][...] = g2[:, f * 128:(f + 1) * 128] * dinv

    return pl.pallas_call(
        body,
        grid=(n8 // BN8,),
        in_specs=[
            pl.BlockSpec((NC, BN8, 128), lambda i: (0, i, 0)),
            pl.BlockSpec((BN8, 128), lambda i: (i, 0)),
            pl.BlockSpec((BN8, 128), lambda i: (i, 0)),
            pl.BlockSpec((128, 512), lambda i: (0, 0)),
            pl.BlockSpec((1, 512), lambda i: (0, 0)),
            pl.BlockSpec((512, 512), lambda i: (0, 0)),
        ],
        out_specs=[pl.BlockSpec((BN8, 128), lambda i: (i, 0))
                   for _ in range(4)],
        out_shape=[jax.ShapeDtypeStruct((n8, 128), jnp.float32)
                   for _ in range(4)],
    )(p1q, g1q, dinvq, W1q, b1q, W2q)


def _tc_final(aggq, g2qs, dinvq, b2q, wlq, bl, n8):
    M = jnp.repeat(jnp.eye(8, dtype=jnp.float32), L, axis=0)

    def body(agg_ref, g0, g1, g2, g3, dinv_ref, b2_ref, wl_ref, m_ref,
             bl_ref, out_ref):
        dinv = dinv_ref[...]
        gs = (g0, g1, g2, g3)
        s = jnp.zeros_like(dinv)
        for f in range(4):
            u = (agg_ref[f, 0] + agg_ref[f, 1] + gs[f][...]) * dinv \
                + b2_ref[f, 0]
            s = s + jnp.maximum(u, 0.0) * wl_ref[f, 0]
        out_ref[...] = (jnp.dot(s, m_ref[...],
                                preferred_element_type=jnp.float32)
                        + bl_ref[0, 0])

    return pl.pallas_call(
        body,
        grid=(n8 // BN8,),
        in_specs=[
            pl.BlockSpec((4, NC, BN8, 128), lambda i: (0, 0, i, 0)),
            pl.BlockSpec((BN8, 128), lambda i: (i, 0)),
            pl.BlockSpec((BN8, 128), lambda i: (i, 0)),
            pl.BlockSpec((BN8, 128), lambda i: (i, 0)),
            pl.BlockSpec((BN8, 128), lambda i: (i, 0)),
            pl.BlockSpec((BN8, 128), lambda i: (i, 0)),
            pl.BlockSpec((4, 1, 128), lambda i: (0, 0, 0)),
            pl.BlockSpec((4, 1, 128), lambda i: (0, 0, 0)),
            pl.BlockSpec((128, 8), lambda i: (0, 0)),
            pl.BlockSpec((1, 1), lambda i: (0, 0)),
        ],
        out_specs=pl.BlockSpec((BN8, 8), lambda i: (i, 0)),
        out_shape=jax.ShapeDtypeStruct((n8, 8), jnp.float32),
    )(aggq, *g2qs, dinvq, b2q.reshape(4, 1, 128), wlq.reshape(4, 1, 128),
      M, bl.reshape(1, 1))


def kernel(x, edge_index, W1, b1, W2, b2, Wl, bl):
    N, IN = x.shape
    E = edge_index.shape[1]

    n_pad = -(-(N + 1) // BN) * BN
    n8 = n_pad // 8
    ew = -(-E // (NW * 2 * CHUNK)) * 2 * CHUNK
    Ep = NW * ew
    e3t = jnp.transpose(edge_index.reshape(2, E // B, B), (1, 0, 2))
    pad_rows = (Ep - E) // B
    tail = jnp.concatenate(
        [jnp.zeros((pad_rows, 1, B), jnp.int32),
         jnp.full((pad_rows, 1, B), N, jnp.int32)], axis=1)
    edges = jnp.concatenate([e3t, tail], axis=0).reshape(-1, B)

    xq = jnp.zeros((n_pad, L), jnp.float32).at[:N, :IN].set(x).reshape(n8, 128)
    W1p = jnp.pad(W1, ((0, L - IN), (0, 0)))
    W1q, b1q, W2q, b2q, wlq = _pack_weights(W1p, b1, W2, b2, Wl)

    degp = _sc_degree_pass(edges, n_pad)
    dinvq, g1q = _tc_prep(degp.reshape(NC, n8, 128), xq, n8)
    g1 = g1q.reshape(n_pad, L)
    p1 = _sc_scatter_pass(edges, [g1], n_pad)[0]
    g2qs = _tc_mid(p1.reshape(NC, n8, 128), g1q, dinvq, W1q, b1q, W2q, n8)
    g2s = [g.reshape(n_pad, L) for g in g2qs]
    agg = _sc_scatter_pass(edges, g2s, n_pad)
    out = _tc_final(agg.reshape(4, NC, n8, 128), g2qs, dinvq, b2q, wlq, bl, n8)
    return out.reshape(n_pad, 1)[:N]
